# Initial kernel scaffold; baseline (speedup 1.0000x reference)
import jax
import jax.numpy as jnp
from jax.experimental import pallas as pl


def kernel(x, edge_index, batch_index, Wg0, bg0, Wg1, bg1, Wg2, bg2, W1, b1, g1, be1, W2, b2, g2, be2, W3, b3):
    raise NotImplementedError("write your pallas kernel here")


# speedup vs baseline: 4.3648x; 4.3648x over previous
import functools

import jax
import jax.numpy as jnp
from jax import lax
from jax.experimental import pallas as pl
from jax.experimental.pallas import tpu as pltpu
from jax.experimental.pallas import tpu_sc as plsc

_N = 10000
_E = 160000
_G = 64
_DIN = 256
_H = 512

_NC, _NS = 2, 16
_CH = 128
_EPAD = 163840
_EPW_DEG = _EPAD // 32
_EPT = _EPAD // _NS
_NROW = 624

_R = 1000
_NB = _N // _R

_f32 = jnp.float32


def _sc_mesh():
    return plsc.VectorSubcoreMesh(
        core_axis_name="c", subcore_axis_name="s",
        num_cores=_NC, num_subcores=_NS)


def _sc_degree(dst_p, ones_tab):
    @functools.partial(
        pl.kernel,
        out_type=jax.ShapeDtypeStruct((_NC * _N, 128), _f32),
        mesh=_sc_mesh(),
        scratch_types=[
            pltpu.VMEM((_CH,), jnp.int32),
            pltpu.VMEM((_CH, 128), _f32),
            pltpu.VMEM_SHARED((_N + 16, 128), _f32),
        ],
    )
    def k(dstr, onr, out, dst_v, ones_v, dacc):
        c = lax.axis_index("c")
        s = lax.axis_index("s")
        pltpu.sync_copy(onr.at[pl.ds(0, _CH)], ones_v)
        r0 = s * _NROW
        base = c * _N
        pltpu.sync_copy(onr.at[pl.ds(r0, _NROW)], dacc.at[pl.ds(r0, _NROW)])

        @pl.when(s == 0)
        def _():
            pltpu.sync_copy(onr.at[pl.ds(9984, 16)], dacc.at[pl.ds(9984, 16)])

        plsc.subcore_barrier()

        def body(i, carry):
            off = (s * _NC + c) * _EPW_DEG + i * _CH
            pltpu.sync_copy(dstr.at[pl.ds(off, _CH)], dst_v)
            pltpu.sync_copy(ones_v, dacc.at[dst_v], add=True)
            return carry

        lax.fori_loop(0, _EPW_DEG // _CH, body, 0)
        plsc.subcore_barrier()
        pltpu.sync_copy(dacc.at[pl.ds(r0, _NROW)], out.at[pl.ds(base + r0, _NROW)])

        @pl.when(s == 0)
        def _():
            pltpu.sync_copy(dacc.at[pl.ds(9984, 16)], out.at[pl.ds(base + 9984, 16)])

    return k(dst_p, ones_tab)


def _sc_gs(hbarf, src_p, dst_p):
    @functools.partial(
        pl.kernel,
        out_type=jax.ShapeDtypeStruct((4 * _N, 128), _f32),
        mesh=_sc_mesh(),
        scratch_types=[
            pltpu.VMEM((_CH,), jnp.int32),
            pltpu.VMEM((_CH,), jnp.int32),
            pltpu.VMEM((_CH, 128), _f32),
            pltpu.VMEM_SHARED((_N + 16, 128), _f32),
            pltpu.SemaphoreType.DMA,
        ],
    )
    def k(hb, srcr, dstr, out, src_v, dst_v, rows_v, acc, sem):
        c = lax.axis_index("c")
        s = lax.axis_index("s")
        r0 = s * _NROW
        for p in range(2):
            j = 2 * c + p
            jrow = j * _N
            pltpu.sync_copy(hb.at[pl.ds(jrow + r0, _NROW)], acc.at[pl.ds(r0, _NROW)])

            @pl.when(s == 0)
            def _():
                pltpu.sync_copy(hb.at[pl.ds(jrow + 9984, 16)], acc.at[pl.ds(9984, 16)])

            plsc.subcore_barrier()

            def body(i, carry):
                off = s * _EPT + i * _CH
                pltpu.sync_copy(srcr.at[pl.ds(off, _CH)], src_v)
                pltpu.sync_copy(dstr.at[pl.ds(off, _CH)], dst_v)
                for kk in range(_CH // 16):
                    sl = pl.ds(kk * 16, 16)
                    src_v[sl] = src_v[sl] + jrow
                pltpu.async_copy(hb.at[src_v], rows_v, sem).wait()
                pltpu.sync_copy(rows_v, acc.at[dst_v], add=True)
                return carry

            lax.fori_loop(0, _EPT // _CH, body, 0)
            plsc.subcore_barrier()
            pltpu.sync_copy(acc.at[pl.ds(r0, _NROW)], out.at[pl.ds(jrow + r0, _NROW)])

            @pl.when(s == 0)
            def _():
                pltpu.sync_copy(acc.at[pl.ds(9984, 16)], out.at[pl.ds(jrow + 9984, 16)])

            plsc.subcore_barrier()

    return k(hbarf, src_p, dst_p)


def _h0_body(x_ref, w_ref, out_ref):
    h = jnp.dot(x_ref[...], w_ref[...], preferred_element_type=_f32)
    for jj in range(4):
        out_ref[jj] = h[:, jj * 128:(jj + 1) * 128]


def _tc_h0(x, w):
    return pl.pallas_call(
        _h0_body,
        grid=(_NB,),
        in_specs=[
            pl.BlockSpec((_R, _DIN), lambda i: (i, 0)),
            pl.BlockSpec((_DIN, _H), lambda i: (0, 0)),
        ],
        out_specs=pl.BlockSpec((4, _R, 128), lambda i: (0, i, 0)),
        out_shape=jax.ShapeDtypeStruct((4, _N, 128), _f32),
    )(x, w)


def _scale_body(degp_ref, h_ref, dinv_ref, out_ref):
    d = degp_ref[...]
    deg = d[0, :, 0] + d[1, :, 0] - 1.0
    dinv = (1.0 / jnp.sqrt(deg))[:, None]
    dinv_ref[...] = dinv
    for jj in range(4):
        out_ref[jj] = h_ref[jj] * dinv


def _tc_scale(degp, h04):
    return pl.pallas_call(
        _scale_body,
        grid=(_NB,),
        in_specs=[
            pl.BlockSpec((_NC, _R, 128), lambda i: (0, i, 0)),
            pl.BlockSpec((4, _R, 128), lambda i: (0, i, 0)),
        ],
        out_specs=[
            pl.BlockSpec((_R, 1), lambda i: (i, 0)),
            pl.BlockSpec((4, _R, 128), lambda i: (0, i, 0)),
        ],
        out_shape=[
            jax.ShapeDtypeStruct((_N, 1), _f32),
            jax.ShapeDtypeStruct((4, _N, 128), _f32),
        ],
    )(degp, h04)


def _layer_body(y_ref, dinv_ref, b_ref, w_ref, out_ref):
    dinv = dinv_ref[...]
    y = jnp.concatenate([y_ref[jj] for jj in range(4)], axis=1)
    a = jnp.maximum(y * dinv + b_ref[...], 0.0)
    acc = jnp.dot(a, w_ref[...], preferred_element_type=_f32)
    hb = acc * dinv
    for jj in range(4):
        out_ref[jj] = hb[:, jj * 128:(jj + 1) * 128]


def _tc_layer(ys, dinv, b_row, w):
    return pl.pallas_call(
        _layer_body,
        grid=(_NB,),
        in_specs=[
            pl.BlockSpec((4, _R, 128), lambda i: (0, i, 0)),
            pl.BlockSpec((_R, 1), lambda i: (i, 0)),
            pl.BlockSpec((1, _H), lambda i: (0, 0)),
            pl.BlockSpec((_H, _H), lambda i: (0, 0)),
        ],
        out_specs=pl.BlockSpec((4, _R, 128), lambda i: (0, i, 0)),
        out_shape=jax.ShapeDtypeStruct((4, _N, 128), _f32),
    )(ys, dinv, b_row, w)


def _pool_body(y_ref, dinv_ref, b_ref, bi_ref, feat_ref, smax_s, ssum_s, cnt_s):
    i = pl.program_id(0)

    @pl.when(i == 0)
    def _():
        smax_s[...] = jnp.full((_G, _H), -jnp.inf, _f32)
        ssum_s[...] = jnp.zeros((_G, _H), _f32)
        cnt_s[...] = jnp.zeros((_G, 1), _f32)

    dinv = dinv_ref[...]
    parts = [y_ref[jj] * dinv + b_ref[jj][None, :] for jj in range(4)]
    h3 = jnp.concatenate(parts, axis=1)
    bi = bi_ref[...][:, 0]
    onehot = (bi[None, :] == lax.broadcasted_iota(jnp.int32, (_G, _R), 0)).astype(_f32)
    ssum_s[...] += jnp.dot(onehot, h3, preferred_element_type=_f32,
                           precision=lax.Precision.HIGHEST)
    cnt_s[...] += jnp.sum(onehot, axis=1, keepdims=True)

    glo = bi_ref[0, 0]
    ghi = bi_ref[_R - 1, 0]

    def gbody(g, carry):
        mask = (bi == g)[:, None]
        m = jnp.max(jnp.where(mask, h3, -jnp.inf), axis=0, keepdims=True)
        cur = smax_s[pl.ds(g, 1), :]
        smax_s[pl.ds(g, 1), :] = jnp.maximum(cur, m)
        return carry

    lax.fori_loop(glo, ghi + 1, gbody, 0)

    @pl.when(i == _NB - 1)
    def _():
        cnt = cnt_s[...]
        feat_ref[:, :_H] = jnp.where(cnt > 0, smax_s[...], 0.0)
        feat_ref[:, _H:] = ssum_s[...] / jnp.maximum(cnt, 1.0)


def _tc_pool(ys, dinv, b4, bi):
    return pl.pallas_call(
        _pool_body,
        grid=(_NB,),
        in_specs=[
            pl.BlockSpec((4, _R, 128), lambda i: (0, i, 0)),
            pl.BlockSpec((_R, 1), lambda i: (i, 0)),
            pl.BlockSpec((4, 128), lambda i: (0, 0)),
            pl.BlockSpec((_R, 1), lambda i: (i, 0)),
        ],
        out_specs=pl.BlockSpec((_G, 2 * _H), lambda i: (0, 0)),
        out_shape=jax.ShapeDtypeStruct((_G, 2 * _H), _f32),
        scratch_shapes=[
            pltpu.VMEM((_G, _H), _f32),
            pltpu.VMEM((_G, _H), _f32),
            pltpu.VMEM((_G, 1), _f32),
        ],
    )(ys, dinv, b4, bi)


def _head_body(f_ref, w1, b1, g1, be1, w2, b2, g2, be2, w3, b3, out_ref):
    def bn(y, g, b):
        m = jnp.mean(y, axis=0, keepdims=True)
        v = jnp.mean((y - m) ** 2, axis=0, keepdims=True)
        return g[...] * (y - m) / jnp.sqrt(v + 1e-5) + b[...]

    y = jnp.dot(f_ref[...], w1[...], preferred_element_type=_f32) + b1[...]
    y = jnp.maximum(bn(y, g1, be1), 0.0)
    y = jnp.dot(y, w2[...], preferred_element_type=_f32) + b2[...]
    y = jnp.maximum(bn(y, g2, be2), 0.0)
    out_ref[...] = jnp.dot(y, w3[...], preferred_element_type=_f32) + b3[...]


def _tc_head(feat, w1, b1, g1, be1, w2, b2, g2, be2, w3p, b3p):
    args = (feat, w1, b1, g1, be1, w2, b2, g2, be2, w3p, b3p)
    return pl.pallas_call(
        _head_body,
        grid=(1,),
        in_specs=[pl.BlockSpec(a.shape, lambda i: tuple(0 for _ in a.shape))
                  for a in args],
        out_specs=pl.BlockSpec((_G, 128), lambda i: (0, 0)),
        out_shape=jax.ShapeDtypeStruct((_G, 128), _f32),
    )(*args)


def kernel(x, edge_index, batch_index, Wg0, bg0, Wg1, bg1, Wg2, bg2,
           W1, b1, g1, be1, W2, b2, g2, be2, W3, b3):
    src = edge_index[0].astype(jnp.int32)
    dst = edge_index[1].astype(jnp.int32)
    pad = _EPAD - _E
    src_p = jnp.concatenate([src, jnp.zeros((pad,), jnp.int32)])
    dst_p = jnp.concatenate([dst, jnp.full((pad,), _N, jnp.int32)])
    ones_tab = jnp.ones((_N, 128), _f32)

    degp = _sc_degree(dst_p, ones_tab).reshape(_NC, _N, 128)
    h04 = _tc_h0(x, Wg0)
    dinv, hbar0 = _tc_scale(degp, h04)

    ys0 = _sc_gs(hbar0.reshape(4 * _N, 128), src_p, dst_p).reshape(4, _N, 128)
    hbar1 = _tc_layer(ys0, dinv, bg0.reshape(1, _H), Wg1)
    ys1 = _sc_gs(hbar1.reshape(4 * _N, 128), src_p, dst_p).reshape(4, _N, 128)
    hbar2 = _tc_layer(ys1, dinv, bg1.reshape(1, _H), Wg2)
    ys2 = _sc_gs(hbar2.reshape(4 * _N, 128), src_p, dst_p).reshape(4, _N, 128)

    feat = _tc_pool(ys2, dinv, bg2.reshape(4, 128),
                    batch_index.reshape(_N, 1).astype(jnp.int32))

    w3p = jnp.pad(W3, ((0, 0), (0, 127)))
    b3p = jnp.pad(b3, (0, 127)).reshape(1, 128)
    y = _tc_head(feat, W1, b1.reshape(1, _H), g1.reshape(1, _H),
                 be1.reshape(1, _H), W2, b2.reshape(1, _H // 2),
                 g2.reshape(1, _H // 2), be2.reshape(1, _H // 2), w3p, b3p)
    return y[:, :1]


# speedup vs baseline: 5.2460x; 1.2019x over previous
import functools

import jax
import jax.numpy as jnp
from jax import lax
from jax.experimental import pallas as pl
from jax.experimental.pallas import tpu as pltpu
from jax.experimental.pallas import tpu_sc as plsc

_N = 10000
_E = 160000
_G = 64
_DIN = 256
_H = 512

_NC, _NS = 2, 16
_CH = 128
_EPAD = 163840
_EPW_DEG = _EPAD // 32
_EPT = _EPAD // _NS
_NROW = 624

_R = 1000
_NB = _N // _R

_f32 = jnp.float32


def _sc_mesh():
    return plsc.VectorSubcoreMesh(
        core_axis_name="c", subcore_axis_name="s",
        num_cores=_NC, num_subcores=_NS)


def _sc_degree(dst_p, ones_tab):
    @functools.partial(
        pl.kernel,
        out_type=jax.ShapeDtypeStruct((_NC * _N, 128), _f32),
        mesh=_sc_mesh(),
        scratch_types=[
            pltpu.VMEM((_CH,), jnp.int32),
            pltpu.VMEM((_CH, 128), _f32),
            pltpu.VMEM_SHARED((_N + 16, 128), _f32),
        ],
    )
    def k(dstr, onr, out, dst_v, ones_v, dacc):
        c = lax.axis_index("c")
        s = lax.axis_index("s")
        pltpu.sync_copy(onr.at[pl.ds(0, _CH)], ones_v)
        r0 = s * _NROW
        base = c * _N
        pltpu.sync_copy(onr.at[pl.ds(r0, _NROW)], dacc.at[pl.ds(r0, _NROW)])

        @pl.when(s == 0)
        def _():
            pltpu.sync_copy(onr.at[pl.ds(9984, 16)], dacc.at[pl.ds(9984, 16)])

        plsc.subcore_barrier()

        def body(i, carry):
            off = (s * _NC + c) * _EPW_DEG + i * _CH
            pltpu.sync_copy(dstr.at[pl.ds(off, _CH)], dst_v)
            pltpu.sync_copy(ones_v, dacc.at[dst_v], add=True)
            return carry

        lax.fori_loop(0, _EPW_DEG // _CH, body, 0)
        plsc.subcore_barrier()
        pltpu.sync_copy(dacc.at[pl.ds(r0, _NROW)], out.at[pl.ds(base + r0, _NROW)])

        @pl.when(s == 0)
        def _():
            pltpu.sync_copy(dacc.at[pl.ds(9984, 16)], out.at[pl.ds(base + 9984, 16)])

    return k(dst_p, ones_tab)


_NCHUNK = _EPT // _CH


def _sc_gs(hbarf, src_p, dst_p):
    @functools.partial(
        pl.kernel,
        out_type=jax.ShapeDtypeStruct((4 * _N, 128), _f32),
        mesh=_sc_mesh(),
        scratch_types=[
            pltpu.VMEM((_EPT,), jnp.int32),
            pltpu.VMEM((_CH,), jnp.int32),
            pltpu.VMEM((_CH,), jnp.int32),
            pltpu.VMEM((_CH, 128), _f32),
            pltpu.VMEM((_CH, 128), _f32),
            pltpu.VMEM_SHARED((_N + 16, 128), _f32),
            pltpu.SemaphoreType.DMA,
            pltpu.SemaphoreType.DMA,
            pltpu.SemaphoreType.DMA,
        ],
    )
    def k(hb, srcr, dstr, out, src_all, dst_v0, dst_v1, rows0, rows1, acc, gsem, ss0, ss1):
        c = lax.axis_index("c")
        s = lax.axis_index("s")
        r0 = s * _NROW
        pltpu.sync_copy(srcr.at[pl.ds(s * _EPT, _EPT)], src_all)
        rows = (rows0, rows1)
        dstv = (dst_v0, dst_v1)
        ssem = (ss0, ss1)
        for p in range(2):
            j = 2 * c + p
            jrow = j * _N
            delta = jrow if p == 0 else _N
            def adj(kk, carry):
                sl = pl.ds(kk * 16, 16)
                src_all[sl] = src_all[sl] + delta
                return carry
            lax.fori_loop(0, _EPT // 16, adj, 0)

            pltpu.sync_copy(hb.at[pl.ds(jrow + r0, _NROW)], acc.at[pl.ds(r0, _NROW)])

            @pl.when(s == 0)
            def _():
                pltpu.sync_copy(hb.at[pl.ds(jrow + 9984, 16)], acc.at[pl.ds(9984, 16)])

            plsc.subcore_barrier()

            def body(o, carry):
                for b in range(2):
                    i = 2 * o + b

                    @pl.when(o > 0)
                    def _():
                        pltpu.make_async_copy(
                            rows[b], acc.at[dstv[b]], ssem[b]).wait()

                    pltpu.sync_copy(dstr.at[pl.ds(s * _EPT + i * _CH, _CH)], dstv[b])
                    pltpu.async_copy(
                        hb.at[src_all.at[pl.ds(i * _CH, _CH)]], rows[b], gsem).wait()
                    pltpu.async_copy(rows[b], acc.at[dstv[b]], ssem[b],
                                     add=True)
                return carry

            lax.fori_loop(0, _NCHUNK // 2, body, 0)
            for b in range(2):
                pltpu.make_async_copy(
                    rows[b], acc.at[dstv[b]], ssem[b]).wait()
            plsc.subcore_barrier()
            pltpu.sync_copy(acc.at[pl.ds(r0, _NROW)], out.at[pl.ds(jrow + r0, _NROW)])

            @pl.when(s == 0)
            def _():
                pltpu.sync_copy(acc.at[pl.ds(9984, 16)], out.at[pl.ds(jrow + 9984, 16)])

            plsc.subcore_barrier()

    return k(hbarf, src_p, dst_p)


def _h0_body(x_ref, w_ref, out_ref):
    h = jnp.dot(x_ref[...], w_ref[...], preferred_element_type=_f32)
    for jj in range(4):
        out_ref[jj] = h[:, jj * 128:(jj + 1) * 128]


def _tc_h0(x, w):
    return pl.pallas_call(
        _h0_body,
        grid=(_NB,),
        in_specs=[
            pl.BlockSpec((_R, _DIN), lambda i: (i, 0)),
            pl.BlockSpec((_DIN, _H), lambda i: (0, 0)),
        ],
        out_specs=pl.BlockSpec((4, _R, 128), lambda i: (0, i, 0)),
        out_shape=jax.ShapeDtypeStruct((4, _N, 128), _f32),
    )(x, w)


def _scale_body(degp_ref, h_ref, dinv_ref, out_ref):
    d = degp_ref[...]
    deg = d[0, :, 0] + d[1, :, 0] - 1.0
    dinv = (1.0 / jnp.sqrt(deg))[:, None]
    dinv_ref[...] = dinv
    for jj in range(4):
        out_ref[jj] = h_ref[jj] * dinv


def _tc_scale(degp, h04):
    return pl.pallas_call(
        _scale_body,
        grid=(_NB,),
        in_specs=[
            pl.BlockSpec((_NC, _R, 128), lambda i: (0, i, 0)),
            pl.BlockSpec((4, _R, 128), lambda i: (0, i, 0)),
        ],
        out_specs=[
            pl.BlockSpec((_R, 1), lambda i: (i, 0)),
            pl.BlockSpec((4, _R, 128), lambda i: (0, i, 0)),
        ],
        out_shape=[
            jax.ShapeDtypeStruct((_N, 1), _f32),
            jax.ShapeDtypeStruct((4, _N, 128), _f32),
        ],
    )(degp, h04)


def _layer_body(y_ref, dinv_ref, b_ref, w_ref, out_ref):
    dinv = dinv_ref[...]
    y = jnp.concatenate([y_ref[jj] for jj in range(4)], axis=1)
    a = jnp.maximum(y * dinv + b_ref[...], 0.0)
    acc = jnp.dot(a, w_ref[...], preferred_element_type=_f32)
    hb = acc * dinv
    for jj in range(4):
        out_ref[jj] = hb[:, jj * 128:(jj + 1) * 128]


def _tc_layer(ys, dinv, b_row, w):
    return pl.pallas_call(
        _layer_body,
        grid=(_NB,),
        in_specs=[
            pl.BlockSpec((4, _R, 128), lambda i: (0, i, 0)),
            pl.BlockSpec((_R, 1), lambda i: (i, 0)),
            pl.BlockSpec((1, _H), lambda i: (0, 0)),
            pl.BlockSpec((_H, _H), lambda i: (0, 0)),
        ],
        out_specs=pl.BlockSpec((4, _R, 128), lambda i: (0, i, 0)),
        out_shape=jax.ShapeDtypeStruct((4, _N, 128), _f32),
    )(ys, dinv, b_row, w)


def _pool_body(y_ref, dinv_ref, b_ref, bi_ref, feat_ref, smax_s, ssum_s, cnt_s):
    i = pl.program_id(0)

    @pl.when(i == 0)
    def _():
        smax_s[...] = jnp.full((_G, _H), -jnp.inf, _f32)
        ssum_s[...] = jnp.zeros((_G, _H), _f32)
        cnt_s[...] = jnp.zeros((_G, 1), _f32)

    dinv = dinv_ref[...]
    parts = [y_ref[jj] * dinv + b_ref[jj][None, :] for jj in range(4)]
    h3 = jnp.concatenate(parts, axis=1)
    bi = bi_ref[...][:, 0]
    onehot = (bi[None, :] == lax.broadcasted_iota(jnp.int32, (_G, _R), 0)).astype(_f32)
    ssum_s[...] += jnp.dot(onehot, h3, preferred_element_type=_f32,
                           precision=lax.Precision.HIGHEST)
    cnt_s[...] += jnp.sum(onehot, axis=1, keepdims=True)

    glo = bi_ref[0, 0]
    ghi = bi_ref[_R - 1, 0]

    def gbody(g, carry):
        mask = (bi == g)[:, None]
        m = jnp.max(jnp.where(mask, h3, -jnp.inf), axis=0, keepdims=True)
        cur = smax_s[pl.ds(g, 1), :]
        smax_s[pl.ds(g, 1), :] = jnp.maximum(cur, m)
        return carry

    lax.fori_loop(glo, ghi + 1, gbody, 0)

    @pl.when(i == _NB - 1)
    def _():
        cnt = cnt_s[...]
        feat_ref[:, :_H] = jnp.where(cnt > 0, smax_s[...], 0.0)
        feat_ref[:, _H:] = ssum_s[...] / jnp.maximum(cnt, 1.0)


def _tc_pool(ys, dinv, b4, bi):
    return pl.pallas_call(
        _pool_body,
        grid=(_NB,),
        in_specs=[
            pl.BlockSpec((4, _R, 128), lambda i: (0, i, 0)),
            pl.BlockSpec((_R, 1), lambda i: (i, 0)),
            pl.BlockSpec((4, 128), lambda i: (0, 0)),
            pl.BlockSpec((_R, 1), lambda i: (i, 0)),
        ],
        out_specs=pl.BlockSpec((_G, 2 * _H), lambda i: (0, 0)),
        out_shape=jax.ShapeDtypeStruct((_G, 2 * _H), _f32),
        scratch_shapes=[
            pltpu.VMEM((_G, _H), _f32),
            pltpu.VMEM((_G, _H), _f32),
            pltpu.VMEM((_G, 1), _f32),
        ],
    )(ys, dinv, b4, bi)


def _head_body(f_ref, w1, b1, g1, be1, w2, b2, g2, be2, w3, b3, out_ref):
    def bn(y, g, b):
        m = jnp.mean(y, axis=0, keepdims=True)
        v = jnp.mean((y - m) ** 2, axis=0, keepdims=True)
        return g[...] * (y - m) / jnp.sqrt(v + 1e-5) + b[...]

    y = jnp.dot(f_ref[...], w1[...], preferred_element_type=_f32) + b1[...]
    y = jnp.maximum(bn(y, g1, be1), 0.0)
    y = jnp.dot(y, w2[...], preferred_element_type=_f32) + b2[...]
    y = jnp.maximum(bn(y, g2, be2), 0.0)
    out_ref[...] = jnp.dot(y, w3[...], preferred_element_type=_f32) + b3[...]


def _tc_head(feat, w1, b1, g1, be1, w2, b2, g2, be2, w3p, b3p):
    args = (feat, w1, b1, g1, be1, w2, b2, g2, be2, w3p, b3p)
    return pl.pallas_call(
        _head_body,
        grid=(1,),
        in_specs=[pl.BlockSpec(a.shape, lambda i: tuple(0 for _ in a.shape))
                  for a in args],
        out_specs=pl.BlockSpec((_G, 128), lambda i: (0, 0)),
        out_shape=jax.ShapeDtypeStruct((_G, 128), _f32),
    )(*args)


def kernel(x, edge_index, batch_index, Wg0, bg0, Wg1, bg1, Wg2, bg2,
           W1, b1, g1, be1, W2, b2, g2, be2, W3, b3):
    src = edge_index[0].astype(jnp.int32)
    dst = edge_index[1].astype(jnp.int32)
    pad = _EPAD - _E
    src_p = jnp.concatenate([src, jnp.zeros((pad,), jnp.int32)])
    dst_p = jnp.concatenate([dst, jnp.full((pad,), _N, jnp.int32)])
    ones_tab = jnp.ones((_N, 128), _f32)

    degp = _sc_degree(dst_p, ones_tab).reshape(_NC, _N, 128)
    h04 = _tc_h0(x, Wg0)
    dinv, hbar0 = _tc_scale(degp, h04)

    ys0 = _sc_gs(hbar0.reshape(4 * _N, 128), src_p, dst_p).reshape(4, _N, 128)
    hbar1 = _tc_layer(ys0, dinv, bg0.reshape(1, _H), Wg1)
    ys1 = _sc_gs(hbar1.reshape(4 * _N, 128), src_p, dst_p).reshape(4, _N, 128)
    hbar2 = _tc_layer(ys1, dinv, bg1.reshape(1, _H), Wg2)
    ys2 = _sc_gs(hbar2.reshape(4 * _N, 128), src_p, dst_p).reshape(4, _N, 128)

    feat = _tc_pool(ys2, dinv, bg2.reshape(4, 128),
                    batch_index.reshape(_N, 1).astype(jnp.int32))

    w3p = jnp.pad(W3, ((0, 0), (0, 127)))
    b3p = jnp.pad(b3, (0, 127)).reshape(1, 128)
    y = _tc_head(feat, W1, b1.reshape(1, _H), g1.reshape(1, _H),
                 be1.reshape(1, _H), W2, b2.reshape(1, _H // 2),
                 g2.reshape(1, _H // 2), be2.reshape(1, _H // 2), w3p, b3p)
    return y[:, :1]
